# Initial kernel scaffold; baseline (speedup 1.0000x reference)
import jax
import jax.numpy as jnp
from jax.experimental import pallas as pl


def kernel(all_init_mats, edge_index, link, n_id, all_sorted_indexes, W_t, W_conv, att_l, att_r, b_conv, W_out):
    raise NotImplementedError("write your pallas kernel here")


# speedup vs baseline: 7.2037x; 7.2037x over previous
import functools

import jax
import jax.numpy as jnp
from jax import lax
from jax.experimental import pallas as pl
from jax.experimental.pallas import tpu as pltpu
from jax.experimental.pallas import tpu_sc as plsc

N = 10000
E = 160000
EA = E + N
D_IN = 256
HID = 128
HEADS = 8
C = 128
OUT = 128
LINKS = 4096
NEG_SLOPE = 0.2

NP = 10240
NT = 320
NB = 32
BPT = NT // NB
K = 32
EP = 170048
TW = 1152
OFFS_LEN = NP + 8
NW = 32


def _mm_body(x_ref, w_ref, o_ref):
    o_ref[...] = jnp.dot(x_ref[...], w_ref[...],
                         preferred_element_type=jnp.float32)


def _matmul(x, w, bm):
    m, kd = x.shape
    bm = min(bm, m)
    _, n = w.shape
    grid = (m // bm,)
    return pl.pallas_call(
        _mm_body,
        grid=grid,
        in_specs=[
            pl.BlockSpec((bm, kd), lambda i: (i, 0)),
            pl.BlockSpec((kd, n), lambda i: (0, 0)),
        ],
        out_specs=pl.BlockSpec((bm, n), lambda i: (i, 0)),
        out_shape=jax.ShapeDtypeStruct((m, n), jnp.float32),
    )(x, w)


def _link_mm_body(z0_ref, z1_ref, w0_ref, w1_ref, o_ref):
    o_ref[...] = (
        jnp.dot(z0_ref[...], w0_ref[...], preferred_element_type=jnp.float32)
        + jnp.dot(z1_ref[...], w1_ref[...], preferred_element_type=jnp.float32)
    )


def _link_matmul(z0, z1, w0t, w1t):
    bm = min(512, LINKS)
    grid = (LINKS // bm,)
    return pl.pallas_call(
        _link_mm_body,
        grid=grid,
        in_specs=[
            pl.BlockSpec((bm, HID), lambda i: (i, 0)),
            pl.BlockSpec((bm, HID), lambda i: (i, 0)),
            pl.BlockSpec((HID, OUT), lambda i: (0, 0)),
            pl.BlockSpec((HID, OUT), lambda i: (0, 0)),
        ],
        out_specs=pl.BlockSpec((bm, OUT), lambda i: (i, 0)),
        out_shape=jax.ShapeDtypeStruct((LINKS, OUT), jnp.float32),
    )(z0, z1, w0t, w1t)


def _sget(ref, i):
    v = plsc.load_gather(ref, [jnp.broadcast_to(i, (16,))])
    return jnp.max(v)


def _sc_layer_body(apply_relu,
                   table_h, srcp_h, dstp_h, valp_h, offs_h, bias_h,
                   xout_h,
                   offs_vt, didx_vt, val_vt,
                   rows_v, drows_v, acc_v, sidx_v,
                   nidx_v, w_v, rec_v, xrow_v, bias_v, sem):
    cid = lax.axis_index("c")
    sid = lax.axis_index("s")
    wid = sid * 2 + cid
    base_node = pl.multiple_of(wid * NT, NT)
    pltpu.sync_copy(offs_h.at[pl.ds(base_node, NT + 8)],
                    offs_vt.at[pl.ds(16, NT + 8)])
    pltpu.sync_copy(bias_h, bias_v)
    lane = lax.iota(jnp.int32, 16)
    zero16 = jnp.zeros((16,), jnp.float32)
    hd_splat = [jnp.full((16,), 16 + h, jnp.int32) for h in range(HEADS)]

    def batch_body(bi, carry):
        n0 = pl.multiple_of(base_node + bi * NB, NB)
        e_lo = _sget(offs_vt, bi * NB + 16)
        e_hi = _sget(offs_vt, bi * NB + NB + 16)
        for j in range(NB // 16):
            nidx_v[pl.ds(j * 16, 16)] = n0 + j * 16 + lane
        pltpu.async_copy(table_h.at[nidx_v], drows_v, sem).wait()

        def zrow(i, c2):
            for g in range(1040 // 16):
                acc_v[i, pl.ds(g * 16, 16)] = zero16
            return c2
        lax.fori_loop(0, NB, zrow, 0)

        e_start = pl.multiple_of(e_lo - lax.rem(e_lo, 8), 8)
        nchunks = (e_hi - e_start + K - 1) // K

        def chunk_body(ci, c3):
            cbase = pl.multiple_of(e_start + ci * K, 8)
            pltpu.sync_copy(srcp_h.at[pl.ds(cbase, K)], sidx_v)
            pltpu.sync_copy(dstp_h.at[pl.ds(cbase, K)],
                            didx_vt.at[pl.ds(16, K)])
            pltpu.sync_copy(valp_h.at[pl.ds(cbase, K)],
                            val_vt.at[pl.ds(16, K)])
            pltpu.async_copy(table_h.at[sidx_v], rows_v, sem).wait()

            def edge_body(e, c4):
                eid = cbase + e
                dstg = _sget(didx_vt, e + 16)
                dstl = dstg - n0
                inr = jnp.logical_and(eid >= e_lo, eid < e_hi)
                dstl_c = jnp.minimum(jnp.maximum(dstl, 0), NB - 1)
                e_splat = jnp.broadcast_to(e + 16, (16,))
                mvec = plsc.load_gather(val_vt, [e_splat])
                t = rows_v[e, pl.ds(1024, 16)]
                u = drows_v[dstl_c, pl.ds(1040, 16)]
                sgt = t + u
                lk = jnp.where(sgt > 0, sgt, NEG_SLOPE * sgt)
                w16 = jnp.exp(lk) * mvec
                w16 = jnp.where(jnp.logical_and(lane < 8, inr), w16, 0.0)
                plsc.addupdate(acc_v.at[dstl_c, pl.ds(1024, 16)], w16)
                w_v[pl.ds(16, 16)] = w16
                for hd in range(HEADS):
                    wsp = plsc.load_gather(w_v, [hd_splat[hd]])
                    for g in range(C // 16):
                        off = hd * C + g * 16
                        plsc.addupdate(acc_v.at[dstl_c, pl.ds(off, 16)],
                                       wsp * rows_v[e, pl.ds(off, 16)])
                return c4
            lax.fori_loop(0, K, edge_body, 0)
            return c3
        lax.fori_loop(0, nchunks, chunk_body, 0)

        def node_body(i, c5):
            den = acc_v[i, pl.ds(1024, 16)]
            rec_v[pl.ds(16, 16)] = 0.125 / (den + 1e-16)
            for g in range(C // 16):
                out16 = bias_v[pl.ds(g * 16, 16)]
                for hd in range(HEADS):
                    rsp = plsc.load_gather(rec_v, [hd_splat[hd]])
                    out16 = out16 + rsp * acc_v[i, pl.ds(hd * C + g * 16, 16)]
                if apply_relu:
                    out16 = jnp.maximum(out16, 0.0)
                xrow_v[i, pl.ds(g * 16, 16)] = out16
            return c5
        lax.fori_loop(0, NB, node_body, 0)
        pltpu.sync_copy(xrow_v, xout_h.at[pl.ds(n0, NB)])
        return carry
    lax.fori_loop(0, BPT, batch_body, 0)


def _sc_layer(table, srcp, dstp, valp, offs, bias, apply_relu):
    mesh = plsc.VectorSubcoreMesh(core_axis_name="c", subcore_axis_name="s")
    kern = pl.kernel(
        functools.partial(_sc_layer_body, apply_relu),
        out_type=jax.ShapeDtypeStruct((NP, HID), jnp.float32),
        mesh=mesh,
        scratch_types=[
            pltpu.VMEM((NT + 24,), jnp.int32),
            pltpu.VMEM((K + 16,), jnp.int32),
            pltpu.VMEM((K + 16,), jnp.float32),
            pltpu.VMEM((K, TW), jnp.float32),
            pltpu.VMEM((NB, TW), jnp.float32),
            pltpu.VMEM((NB, 1040), jnp.float32),
            pltpu.VMEM((K,), jnp.int32),
            pltpu.VMEM((NB,), jnp.int32),
            pltpu.VMEM((32,), jnp.float32),
            pltpu.VMEM((32,), jnp.float32),
            pltpu.VMEM((NB, HID), jnp.float32),
            pltpu.VMEM((HID,), jnp.float32),
            pltpu.SemaphoreType.DMA,
        ],
        compiler_params=pltpu.CompilerParams(needs_layout_passes=False),
    )
    return kern(table, srcp, dstp, valp, offs, bias)


def _sc_link_body(x_h, l0_h, l1_h, z0_h, z1_h, idx_v, rows_v, sem):
    cid = lax.axis_index("c")
    sid = lax.axis_index("s")
    wid = sid * 2 + cid
    base = pl.multiple_of(wid * (LINKS // NW), LINKS // NW)
    pltpu.sync_copy(l0_h.at[pl.ds(base, LINKS // NW)], idx_v)
    pltpu.async_copy(x_h.at[idx_v], rows_v, sem).wait()
    pltpu.sync_copy(rows_v, z0_h.at[pl.ds(base, LINKS // NW)])
    pltpu.sync_copy(l1_h.at[pl.ds(base, LINKS // NW)], idx_v)
    pltpu.async_copy(x_h.at[idx_v], rows_v, sem).wait()
    pltpu.sync_copy(rows_v, z1_h.at[pl.ds(base, LINKS // NW)])


def _sc_link_gather(x, l0, l1):
    mesh = plsc.VectorSubcoreMesh(core_axis_name="c", subcore_axis_name="s")
    kern = pl.kernel(
        _sc_link_body,
        out_type=(
            jax.ShapeDtypeStruct((LINKS, HID), jnp.float32),
            jax.ShapeDtypeStruct((LINKS, HID), jnp.float32),
        ),
        mesh=mesh,
        scratch_types=[
            pltpu.VMEM((LINKS // NW,), jnp.int32),
            pltpu.VMEM((LINKS // NW, HID), jnp.float32),
            pltpu.SemaphoreType.DMA,
        ],
        compiler_params=pltpu.CompilerParams(needs_layout_passes=False),
    )
    return kern(x, l0, l1)


def _build_wext(w_conv_l, att_l_l, att_r_l):
    in_dim = w_conv_l.shape[1]
    wct = w_conv_l.T
    wc3 = w_conv_l.reshape(HEADS, C, in_dim)
    wa_l = jnp.einsum("hci,hc->ih", wc3, att_l_l[0])
    wa_r = jnp.einsum("hci,hc->ih", wc3, att_r_l[0])
    z8 = jnp.zeros((in_dim, 8), jnp.float32)
    ztail = jnp.zeros((in_dim, TW - 1048), jnp.float32)
    return jnp.concatenate([wct, wa_l, z8, wa_r, ztail], axis=1)


def kernel(all_init_mats, edge_index, link, n_id, all_sorted_indexes,
           W_t, W_conv, att_l, att_r, b_conv, W_out):
    mats = all_init_mats[all_sorted_indexes[n_id]]
    mats = jnp.pad(mats, ((0, NP - N), (0, 0)))

    src, dst = edge_index[0], edge_index[1]
    self_idx = jnp.arange(N, dtype=jnp.int32)
    src_all = jnp.concatenate([src, self_idx])
    dst_all = jnp.concatenate([dst, self_idx])
    valid = jnp.concatenate([(src != dst), jnp.ones((N,), bool)])
    perm = jnp.argsort(dst_all)
    src_p = jnp.concatenate(
        [src_all[perm], jnp.zeros((EP - EA,), jnp.int32)])
    dst_p_real = dst_all[perm]
    dst_p = jnp.concatenate([dst_p_real, jnp.zeros((EP - EA,), jnp.int32)])
    val_p = jnp.concatenate(
        [valid[perm].astype(jnp.float32), jnp.zeros((EP - EA,), jnp.float32)])
    offs = jnp.searchsorted(
        dst_p_real, jnp.arange(NP + 8, dtype=jnp.int32), side="left"
    ).astype(jnp.int32)

    wext0 = _build_wext(W_conv[0], att_l[0], att_r[0])
    wfused0 = W_t.T @ wext0
    wext1 = _build_wext(W_conv[1], att_l[1], att_r[1])

    table0 = _matmul(mats, wfused0, bm=256)
    x1 = _sc_layer(table0, src_p, dst_p, val_p, offs, b_conv[0],
                   apply_relu=True)
    table1 = _matmul(x1, wext1, bm=256)
    x2 = _sc_layer(table1, src_p, dst_p, val_p, offs, b_conv[1],
                   apply_relu=False)

    z0, z1 = _sc_link_gather(x2, link[0], link[1])
    w0t = W_out[:, :HID].T
    w1t = W_out[:, HID:].T
    return _link_matmul(z0, z1, w0t, w1t)
